# baseline (speedup 1.0000x reference)
import jax
import jax.numpy as jnp
from jax.experimental import pallas as pl
from jax.experimental.pallas import tpu as pltpu

CTX = 256
N = 20000
B = 8
STRIPS = 4
STRIP_ROWS = B * N // STRIPS
BLOCK = 1600
STEPS = STRIP_ROWS // BLOCK


def _matmul_kernel(x_ref, w_ref, bias_ref, o0, o1, o2, o3):
    w = w_ref[...]
    bias = bias_ref[...]
    for s, o_ref in enumerate((o0, o1, o2, o3)):
        x = x_ref[s].astype(jnp.bfloat16)
        y = jnp.dot(x, w, preferred_element_type=jnp.float32)
        o_ref[...] = y + bias


def _argmax_kernel(s_ref, idx_ref):
    s = s_ref[0]
    m = jnp.max(s)
    ii = jax.lax.broadcasted_iota(jnp.int32, s.shape, 1)
    idx = jnp.min(jnp.where(s == m, ii, N))
    idx_ref[...] = jnp.full((1, 1, 1), idx, jnp.int32)


def _gather_kernel(idx_ref, xrow_ref, w_ref, bias_ref, off_ref):
    xr = xrow_ref[0]
    y = jnp.dot(xr, w_ref[...], preferred_element_type=jnp.float32)
    off_ref[0] = y + bias_ref[...]


@jax.jit
def kernel(x_out, ref_scores, W, b):
    w_bf = W.astype(jnp.bfloat16)
    bias = b.reshape(1, 4)
    x3 = x_out.reshape(STRIPS, STRIP_ROWS, CTX)

    strips = pl.pallas_call(
        _matmul_kernel,
        grid=(STEPS,),
        in_specs=[pl.BlockSpec((STRIPS, BLOCK, CTX), lambda i: (0, i, 0)),
                  pl.BlockSpec((CTX, 4), lambda i: (0, 0)),
                  pl.BlockSpec((1, 4), lambda i: (0, 0))],
        out_specs=[pl.BlockSpec((BLOCK, 4), lambda i: (i, 0))] * STRIPS,
        out_shape=[jax.ShapeDtypeStruct((STRIP_ROWS, 4), jnp.float32)] * STRIPS,
    )(x3, w_bf, bias)
    out = jnp.concatenate(strips, axis=0).reshape(B, N, 4)

    idx = pl.pallas_call(
        _argmax_kernel,
        grid=(B,),
        in_specs=[pl.BlockSpec((1, 1, N), lambda bi: (bi, 0, 0))],
        out_specs=pl.BlockSpec((1, 1, 1), lambda bi: (bi, 0, 0)),
        out_shape=jax.ShapeDtypeStruct((B, 1, 1), jnp.int32),
    )(ref_scores.reshape(B, 1, N))
    idx_flat = idx.reshape(B)

    off = pl.pallas_call(
        _gather_kernel,
        grid_spec=pltpu.PrefetchScalarGridSpec(
            num_scalar_prefetch=1,
            grid=(B,),
            in_specs=[
                pl.BlockSpec((1, 1, CTX),
                             lambda bi, idx_p: (bi * N + idx_p[bi], 0, 0)),
                pl.BlockSpec((CTX, 4), lambda bi, idx_p: (0, 0)),
                pl.BlockSpec((1, 4), lambda bi, idx_p: (0, 0)),
            ],
            out_specs=pl.BlockSpec((1, 1, 4),
                                   lambda bi, idx_p: (bi, 0, 0)),
        ),
        out_shape=jax.ShapeDtypeStruct((B, 1, 4), jnp.float32),
    )(idx_flat, x_out.reshape(B * N, 1, CTX), W, bias)

    rows = jnp.arange(B, dtype=jnp.int32)
    slice_inds = jnp.stack([rows, idx_flat], axis=1)
    return (off.reshape(B, 4), out, slice_inds)


# speedup vs baseline: 4.2890x; 4.2890x over previous
import jax
import jax.numpy as jnp
from jax.experimental import pallas as pl
from jax.experimental.pallas import tpu as pltpu

CTX = 256
N = 20000
B = 8
STRIPS = 4
STRIP_ROWS = B * N // STRIPS
BLOCK = 1600
STEPS = STRIP_ROWS // BLOCK


def _matmul_kernel(x_ref, w_ref, bias_ref, o0, o1, o2, o3):
    w = w_ref[...]
    bias = bias_ref[...]
    for s, o_ref in enumerate((o0, o1, o2, o3)):
        x = x_ref[s].astype(jnp.bfloat16)
        y = jnp.dot(x, w, preferred_element_type=jnp.float32)
        o_ref[...] = y + bias


def _argmax_kernel(s_ref, idx_ref):
    s = s_ref[...]
    m = jnp.max(s, axis=1, keepdims=True)
    ii = jax.lax.broadcasted_iota(jnp.int32, s.shape, 1)
    idx_ref[...] = jnp.min(jnp.where(s == m, ii, N), axis=1, keepdims=True)


def _gather_kernel(idx_ref, x8_ref, w_ref, bias_ref, off_ref):
    bi = pl.program_id(0)
    local = (bi * N + idx_ref[bi]) % 8
    x8 = x8_ref[...].astype(jnp.bfloat16)
    y = jnp.dot(x8, w_ref[...], preferred_element_type=jnp.float32)
    rows = jax.lax.broadcasted_iota(jnp.int32, (8, 1), 0)
    off_ref[0] = jnp.sum(jnp.where(rows == local, y, 0.0), axis=0,
                         keepdims=True) + bias_ref[...]


@jax.jit
def kernel(x_out, ref_scores, W, b):
    w_bf = W.astype(jnp.bfloat16)
    bias = b.reshape(1, 4)
    x3 = x_out.reshape(STRIPS, STRIP_ROWS, CTX)

    strips = pl.pallas_call(
        _matmul_kernel,
        grid=(STEPS,),
        in_specs=[pl.BlockSpec((STRIPS, BLOCK, CTX), lambda i: (0, i, 0)),
                  pl.BlockSpec((CTX, 4), lambda i: (0, 0)),
                  pl.BlockSpec((1, 4), lambda i: (0, 0))],
        out_specs=[pl.BlockSpec((BLOCK, 4), lambda i: (i, 0))] * STRIPS,
        out_shape=[jax.ShapeDtypeStruct((STRIP_ROWS, 4), jnp.float32)] * STRIPS,
    )(x3, w_bf, bias)
    out = jnp.concatenate(strips, axis=0).reshape(B, N, 4)

    idx = pl.pallas_call(
        _argmax_kernel,
        in_specs=[pl.BlockSpec((B, N), lambda: (0, 0))],
        out_specs=pl.BlockSpec((B, 1), lambda: (0, 0)),
        out_shape=jax.ShapeDtypeStruct((B, 1), jnp.int32),
    )(ref_scores)
    idx_flat = idx.reshape(B)

    off = pl.pallas_call(
        _gather_kernel,
        grid_spec=pltpu.PrefetchScalarGridSpec(
            num_scalar_prefetch=1,
            grid=(B,),
            in_specs=[
                pl.BlockSpec((8, CTX),
                             lambda bi, idx_p: ((bi * N + idx_p[bi]) // 8, 0)),
                pl.BlockSpec((CTX, 4), lambda bi, idx_p: (0, 0)),
                pl.BlockSpec((1, 4), lambda bi, idx_p: (0, 0)),
            ],
            out_specs=pl.BlockSpec((1, 1, 4), lambda bi, idx_p: (bi, 0, 0)),
        ),
        out_shape=jax.ShapeDtypeStruct((B, 1, 4), jnp.float32),
    )(idx_flat, x_out.reshape(B * N, CTX), W, bias)

    rows = jnp.arange(B, dtype=jnp.int32)
    slice_inds = jnp.stack([rows, idx_flat], axis=1)
    return (off.reshape(B, 4), out, slice_inds)


# speedup vs baseline: 8.8437x; 2.0619x over previous
import jax
import jax.numpy as jnp
from jax.experimental import pallas as pl
from jax.experimental.pallas import tpu as pltpu

CTX = 256
N = 20000
B = 8
BLOCK = 6400


def _matmul_kernel(x_ref, w_ref, bias_ref, out_ref):
    x = x_ref[...].astype(jnp.bfloat16)
    y_t = jax.lax.dot_general(w_ref[...], x, (((0,), (1,)), ((), ())),
                              preferred_element_type=jnp.float32)
    out_ref[...] = y_t + bias_ref[...]


def _argmax_kernel(s_ref, idx_ref):
    s = s_ref[...]
    m = jnp.max(s, axis=1, keepdims=True)
    ii = jax.lax.broadcasted_iota(jnp.int32, s.shape, 1)
    idx_ref[...] = jnp.min(jnp.where(s == m, ii, N), axis=1, keepdims=True)


def _gather_kernel(idx_ref, x8_ref, w_ref, bias_ref, off_ref):
    bi = pl.program_id(0)
    local = (bi * N + idx_ref[bi]) % 8
    x8 = x8_ref[...].astype(jnp.bfloat16)
    y = jnp.dot(x8, w_ref[...], preferred_element_type=jnp.float32)
    rows = jax.lax.broadcasted_iota(jnp.int32, (8, 1), 0)
    off_ref[0] = jnp.sum(jnp.where(rows == local, y, 0.0), axis=0,
                         keepdims=True) + bias_ref[...]


@jax.jit
def kernel(x_out, ref_scores, W, b):
    w_bf = W.astype(jnp.bfloat16)
    bias = b.reshape(1, 4)
    x2 = x_out.reshape(B * N, CTX)

    out_t = pl.pallas_call(
        _matmul_kernel,
        grid=(B * N // BLOCK,),
        in_specs=[pl.BlockSpec((BLOCK, CTX), lambda i: (i, 0)),
                  pl.BlockSpec((CTX, 4), lambda i: (0, 0)),
                  pl.BlockSpec((4, 1), lambda i: (0, 0))],
        out_specs=pl.BlockSpec((4, BLOCK), lambda i: (0, i)),
        out_shape=jax.ShapeDtypeStruct((4, B * N), jnp.float32),
    )(x2, w_bf, b.reshape(4, 1))
    out = out_t.T.reshape(B, N, 4)

    idx = pl.pallas_call(
        _argmax_kernel,
        in_specs=[pl.BlockSpec((B, N), lambda: (0, 0))],
        out_specs=pl.BlockSpec((B, 1), lambda: (0, 0)),
        out_shape=jax.ShapeDtypeStruct((B, 1), jnp.int32),
    )(ref_scores)
    idx_flat = idx.reshape(B)

    off = pl.pallas_call(
        _gather_kernel,
        grid_spec=pltpu.PrefetchScalarGridSpec(
            num_scalar_prefetch=1,
            grid=(B,),
            in_specs=[
                pl.BlockSpec((8, CTX),
                             lambda bi, idx_p: ((bi * N + idx_p[bi]) // 8, 0)),
                pl.BlockSpec((CTX, 4), lambda bi, idx_p: (0, 0)),
                pl.BlockSpec((1, 4), lambda bi, idx_p: (0, 0)),
            ],
            out_specs=pl.BlockSpec((1, 1, 4), lambda bi, idx_p: (bi, 0, 0)),
        ),
        out_shape=jax.ShapeDtypeStruct((B, 1, 4), jnp.float32),
    )(idx_flat, x_out.reshape(B * N, CTX), W, bias)

    rows = jnp.arange(B, dtype=jnp.int32)
    slice_inds = jnp.stack([rows, idx_flat], axis=1)
    return (off.reshape(B, 4), out, slice_inds)


# speedup vs baseline: 8.9106x; 1.0076x over previous
import jax
import jax.numpy as jnp
from jax.experimental import pallas as pl
from jax.experimental.pallas import tpu as pltpu

CTX = 256
N = 20000
B = 8
BLOCK = 16000


def _matmul_kernel(x_ref, w_ref, bias_ref, out_ref):
    x = x_ref[...].astype(jnp.bfloat16)
    y_t = jax.lax.dot_general(w_ref[...], x, (((0,), (1,)), ((), ())),
                              preferred_element_type=jnp.float32)
    out_ref[...] = y_t + bias_ref[...]


def _argmax_kernel(s_ref, idx_ref):
    s = s_ref[...]
    m = jnp.max(s, axis=1, keepdims=True)
    ii = jax.lax.broadcasted_iota(jnp.int32, s.shape, 1)
    idx_ref[...] = jnp.min(jnp.where(s == m, ii, N), axis=1, keepdims=True)


def _gather_kernel(idx_ref, x8_ref, w_ref, bias_ref, off_ref):
    bi = pl.program_id(0)
    local = (bi * N + idx_ref[bi]) % 8
    x8 = x8_ref[...].astype(jnp.bfloat16)
    y = jnp.dot(x8, w_ref[...], preferred_element_type=jnp.float32)
    rows = jax.lax.broadcasted_iota(jnp.int32, (8, 1), 0)
    off_ref[0] = jnp.sum(jnp.where(rows == local, y, 0.0), axis=0,
                         keepdims=True) + bias_ref[...]


@jax.jit
def kernel(x_out, ref_scores, W, b):
    w_bf = W.astype(jnp.bfloat16)
    bias = b.reshape(1, 4)
    x2 = x_out.reshape(B * N, CTX)

    out_t = pl.pallas_call(
        _matmul_kernel,
        grid=(B * N // BLOCK,),
        in_specs=[pl.BlockSpec((BLOCK, CTX), lambda i: (i, 0)),
                  pl.BlockSpec((CTX, 4), lambda i: (0, 0)),
                  pl.BlockSpec((4, 1), lambda i: (0, 0))],
        out_specs=pl.BlockSpec((4, BLOCK), lambda i: (0, i)),
        out_shape=jax.ShapeDtypeStruct((4, B * N), jnp.float32),
    )(x2, w_bf, b.reshape(4, 1))
    out = out_t.T.reshape(B, N, 4)

    idx = pl.pallas_call(
        _argmax_kernel,
        in_specs=[pl.BlockSpec((B, N), lambda: (0, 0))],
        out_specs=pl.BlockSpec((B, 1), lambda: (0, 0)),
        out_shape=jax.ShapeDtypeStruct((B, 1), jnp.int32),
    )(ref_scores)
    idx_flat = idx.reshape(B)

    off = pl.pallas_call(
        _gather_kernel,
        grid_spec=pltpu.PrefetchScalarGridSpec(
            num_scalar_prefetch=1,
            grid=(B,),
            in_specs=[
                pl.BlockSpec((8, CTX),
                             lambda bi, idx_p: ((bi * N + idx_p[bi]) // 8, 0)),
                pl.BlockSpec((CTX, 4), lambda bi, idx_p: (0, 0)),
                pl.BlockSpec((1, 4), lambda bi, idx_p: (0, 0)),
            ],
            out_specs=pl.BlockSpec((1, 1, 4), lambda bi, idx_p: (bi, 0, 0)),
        ),
        out_shape=jax.ShapeDtypeStruct((B, 1, 4), jnp.float32),
    )(idx_flat, x_out.reshape(B * N, CTX), W, bias)

    rows = jnp.arange(B, dtype=jnp.int32)
    slice_inds = jnp.stack([rows, idx_flat], axis=1)
    return (off.reshape(B, 4), out, slice_inds)
